# Initial kernel scaffold; baseline (speedup 1.0000x reference)
import jax
import jax.numpy as jnp
from jax.experimental import pallas as pl


def kernel(x, edge_index, edge_r, node_inv, Wi1, bi1, Wi2, bi2, Wi3, bi3, Wa1, ba1, Wa2, ba2, Wa3, ba3):
    raise NotImplementedError("write your pallas kernel here")


# speedup vs baseline: 3.1147x; 3.1147x over previous
import functools

import jax
import jax.numpy as jnp
from jax import lax
from jax.experimental import pallas as pl
from jax.experimental.pallas import tpu as pltpu
from jax.experimental.pallas import tpu_sc as plsc

NC = 2
NS = 16
NW = NC * NS
CH = 128


def _leaky(v, slope=0.01):
    return jnp.where(v >= 0, v, slope * v)


def _mlp3(v, W1, b1, W2, b2, W3, b3):
    v = _leaky(jnp.dot(v, W1, preferred_element_type=jnp.float32) + b1)
    v = _leaky(jnp.dot(v, W2, preferred_element_type=jnp.float32) + b2)
    return jnp.dot(v, W3, preferred_element_type=jnp.float32) + b3


def _prep_body(x_ref, W1, b1, W2, b2, W3, b3, out_ref):
    xa = x_ref[...]
    out_ref[0] = xa
    out_ref[1] = _mlp3(xa, W1[...], b1[...], W2[...], b2[...], W3[...], b3[...])


def _idx_body(n, src_ref, r_ref, out_ref):
    out_ref[...] = src_ref[...] + n * (r_ref[...] == 1).astype(jnp.int32)


def _final_body(sum_ref, deg_ref, inv_ref,
                Wa1, ba1, Wa2, ba2, Wa3, ba3,
                Wi1, bi1, Wi2, bi2, Wi3, bi3, out_ref):
    ssum = sum_ref[0] + sum_ref[1]
    deg = deg_ref[0] + deg_ref[1]
    neigh = ssum / jnp.maximum(deg, 1.0)
    a = _mlp3(neigh, Wa1[...], ba1[...], Wa2[...], ba2[...], Wa3[...], ba3[...])
    b = _mlp3(a, Wi1[...], bi1[...], Wi2[...], bi2[...], Wi3[...], bi3[...])
    out_ref[...] = jnp.where(inv_ref[...] == 1, b, a)


def kernel(x, edge_index, edge_r, node_inv,
           Wi1, bi1, Wi2, bi2, Wi3, bi3,
           Wa1, ba1, Wa2, ba2, Wa3, ba3):
    N, D = x.shape
    H = Wi1.shape[1]
    E = edge_index.shape[1]
    src = edge_index[0]
    dst = edge_index[1]

    bi1r, bi2r, bi3r = bi1.reshape(1, H), bi2.reshape(1, H), bi3.reshape(1, D)
    ba1r, ba2r, ba3r = ba1.reshape(1, H), ba2.reshape(1, H), ba3.reshape(1, D)

    BN = 1000
    grid = N // BN
    table = pl.pallas_call(
        _prep_body,
        grid=(grid,),
        in_specs=[
            pl.BlockSpec((BN, D), lambda i: (i, 0)),
            pl.BlockSpec((D, H), lambda i: (0, 0)),
            pl.BlockSpec((1, H), lambda i: (0, 0)),
            pl.BlockSpec((H, H), lambda i: (0, 0)),
            pl.BlockSpec((1, H), lambda i: (0, 0)),
            pl.BlockSpec((H, D), lambda i: (0, 0)),
            pl.BlockSpec((1, D), lambda i: (0, 0)),
        ],
        out_specs=pl.BlockSpec((2, BN, D), lambda i: (0, i, 0)),
        out_shape=jax.ShapeDtypeStruct((2, N, D), jnp.float32),
    )(x, Wi1, bi1r, Wi2, bi2r, Wi3, bi3r)
    table2n = table.reshape(2 * N, D)

    ER = E // 128
    idx = pl.pallas_call(
        functools.partial(_idx_body, N),
        out_shape=jax.ShapeDtypeStruct((ER, 128), jnp.int32),
    )(src.reshape(ER, 128), edge_r.reshape(ER, 128)).reshape(E)

    nchunk = -(-E // (NW * CH))
    if nchunk % 2:
        nchunk += 1
    epw = nchunk * CH
    epad = NW * epw
    npad = -(-(N + 1) // 128) * 128
    pad = epad - E
    idx_p = jnp.concatenate([idx, jnp.zeros((pad,), jnp.int32)])
    dst_p = jnp.concatenate([dst, jnp.full((pad,), N, jnp.int32)])
    zrows = jnp.zeros((128, D), jnp.float32)
    orows = jnp.ones((CH, D), jnp.float32)
    zpt = npad // NS

    def zero_acc(src_stage, acc, s):
        off = 0
        while off < zpt:
            ln = min(128, zpt - off)
            pltpu.sync_copy(src_stage.at[pl.ds(0, ln)],
                            acc.at[pl.ds(s * zpt + off, ln)])
            off += ln

    def export_acc(acc, stage, out_hbm, c, s):
        off = 0
        while off < zpt:
            ln = min(128, zpt - off)
            pltpu.sync_copy(acc.at[pl.ds(s * zpt + off, ln)],
                            stage.at[pl.ds(0, ln)])
            pltpu.sync_copy(stage.at[pl.ds(0, ln)],
                            out_hbm.at[c, pl.ds(s * zpt + off, ln)])
            off += ln

    def sum_body(table, idxp, dstp, zrows_r, sum_out,
                 idx0, idx1, dst0, dst1, rows0, rows1, acc, sem0, sem1):
        c = lax.axis_index("c")
        s = lax.axis_index("s")
        pltpu.sync_copy(zrows_r, rows0)
        zero_acc(rows0, acc, s)
        plsc.subcore_barrier()

        ebase = (c * NS + s) * epw

        def chunk(g, carry):
            eb = ebase + g * CH
            pltpu.sync_copy(idxp.at[pl.ds(eb, CH)], idx0)
            pltpu.sync_copy(dstp.at[pl.ds(eb, CH)], dst0)
            pltpu.async_copy(table.at[idx0], rows0, sem0).wait()
            pltpu.sync_copy(rows0, acc.at[dst0], add=True)
            return carry

        lax.fori_loop(0, nchunk, chunk, 0, unroll=False)
        plsc.subcore_barrier()
        export_acc(acc, rows0, sum_out, c, s)

    def deg_body(dstp, zrows_r, orows_r, deg_out,
                 dst0, dst1, rows0, acc, sem0):
        c = lax.axis_index("c")
        s = lax.axis_index("s")
        pltpu.sync_copy(zrows_r, rows0)
        zero_acc(rows0, acc, s)
        plsc.subcore_barrier()
        pltpu.sync_copy(orows_r, rows0)

        ebase = (c * NS + s) * epw

        def chunk(g, carry):
            eb = ebase + g * CH
            pltpu.sync_copy(dstp.at[pl.ds(eb, CH)], dst0)
            pltpu.sync_copy(rows0, acc.at[dst0], add=True)
            return carry

        lax.fori_loop(0, nchunk, chunk, 0, unroll=False)
        plsc.subcore_barrier()
        export_acc(acc, rows0, deg_out, c, s)

    mesh = plsc.VectorSubcoreMesh(
        core_axis_name="c", subcore_axis_name="s",
        num_cores=NC, num_subcores=NS)
    sum_call = pl.kernel(
        sum_body,
        out_type=jax.ShapeDtypeStruct((NC, npad, D), jnp.float32),
        mesh=mesh,
        scratch_types=[
            pltpu.VMEM((CH,), jnp.int32),
            pltpu.VMEM((CH,), jnp.int32),
            pltpu.VMEM((CH,), jnp.int32),
            pltpu.VMEM((CH,), jnp.int32),
            pltpu.VMEM((CH, D), jnp.float32),
            pltpu.VMEM((CH, D), jnp.float32),
            pltpu.VMEM_SHARED((npad, D), jnp.float32),
            pltpu.SemaphoreType.DMA,
            pltpu.SemaphoreType.DMA,
        ],
    )
    sum_p = sum_call(table2n, idx_p, dst_p, zrows)

    deg_call = pl.kernel(
        deg_body,
        out_type=jax.ShapeDtypeStruct((NC, npad, D), jnp.float32),
        mesh=mesh,
        scratch_types=[
            pltpu.VMEM((CH,), jnp.int32),
            pltpu.VMEM((CH,), jnp.int32),
            pltpu.VMEM((CH, D), jnp.float32),
            pltpu.VMEM_SHARED((npad, D), jnp.float32),
            pltpu.SemaphoreType.DMA,
        ],
    )
    deg_p = deg_call(dst_p, zrows, orows)

    sum_p = sum_p[:, :N]
    deg_p = deg_p[:, :N, 0:1]

    inv2d = node_inv.reshape(N, 1)
    out = pl.pallas_call(
        _final_body,
        grid=(grid,),
        in_specs=[
            pl.BlockSpec((NC, BN, D), lambda i: (0, i, 0)),
            pl.BlockSpec((NC, BN, 1), lambda i: (0, i, 0)),
            pl.BlockSpec((BN, 1), lambda i: (i, 0)),
            pl.BlockSpec((D, H), lambda i: (0, 0)),
            pl.BlockSpec((1, H), lambda i: (0, 0)),
            pl.BlockSpec((H, H), lambda i: (0, 0)),
            pl.BlockSpec((1, H), lambda i: (0, 0)),
            pl.BlockSpec((H, D), lambda i: (0, 0)),
            pl.BlockSpec((1, D), lambda i: (0, 0)),
            pl.BlockSpec((D, H), lambda i: (0, 0)),
            pl.BlockSpec((1, H), lambda i: (0, 0)),
            pl.BlockSpec((H, H), lambda i: (0, 0)),
            pl.BlockSpec((1, H), lambda i: (0, 0)),
            pl.BlockSpec((H, D), lambda i: (0, 0)),
            pl.BlockSpec((1, D), lambda i: (0, 0)),
        ],
        out_specs=pl.BlockSpec((BN, D), lambda i: (i, 0)),
        out_shape=jax.ShapeDtypeStruct((N, D), jnp.float32),
    )(sum_p, deg_p, inv2d,
      Wa1, ba1r, Wa2, ba2r, Wa3, ba3r,
      Wi1, bi1r, Wi2, bi2r, Wi3, bi3r)
    return out


# speedup vs baseline: 3.8353x; 1.2313x over previous
import functools

import jax
import jax.numpy as jnp
from jax import lax
from jax.experimental import pallas as pl
from jax.experimental.pallas import tpu as pltpu
from jax.experimental.pallas import tpu_sc as plsc

NC = 2
NS = 16
NW = NC * NS
CH = 128


def _leaky(v, slope=0.01):
    return jnp.where(v >= 0, v, slope * v)


def _mlp3(v, W1, b1, W2, b2, W3, b3):
    v = _leaky(jnp.dot(v, W1, preferred_element_type=jnp.float32) + b1)
    v = _leaky(jnp.dot(v, W2, preferred_element_type=jnp.float32) + b2)
    return jnp.dot(v, W3, preferred_element_type=jnp.float32) + b3


def _prep_body(x_ref, W1, b1, W2, b2, W3, b3, out_ref):
    xa = x_ref[...]
    out_ref[0] = xa
    out_ref[1] = _mlp3(xa, W1[...], b1[...], W2[...], b2[...], W3[...], b3[...])


def _idx_body(n, src_ref, r_ref, out_ref):
    out_ref[...] = src_ref[...] + n * (r_ref[...] == 1).astype(jnp.int32)


def _final_body(sum_ref, deg_ref, inv_ref,
                Wa1, ba1, Wa2, ba2, Wa3, ba3,
                Wi1, bi1, Wi2, bi2, Wi3, bi3, out_ref):
    ssum = sum_ref[0] + sum_ref[1]
    deg = deg_ref[0] + deg_ref[1]
    neigh = ssum / jnp.maximum(deg, 1.0)
    a = _mlp3(neigh, Wa1[...], ba1[...], Wa2[...], ba2[...], Wa3[...], ba3[...])
    b = _mlp3(a, Wi1[...], bi1[...], Wi2[...], bi2[...], Wi3[...], bi3[...])
    out_ref[...] = jnp.where(inv_ref[...] == 1, b, a)


def kernel(x, edge_index, edge_r, node_inv,
           Wi1, bi1, Wi2, bi2, Wi3, bi3,
           Wa1, ba1, Wa2, ba2, Wa3, ba3):
    N, D = x.shape
    H = Wi1.shape[1]
    E = edge_index.shape[1]
    src = edge_index[0]
    dst = edge_index[1]

    bi1r, bi2r, bi3r = bi1.reshape(1, H), bi2.reshape(1, H), bi3.reshape(1, D)
    ba1r, ba2r, ba3r = ba1.reshape(1, H), ba2.reshape(1, H), ba3.reshape(1, D)

    BN = 1000
    grid = N // BN
    table = pl.pallas_call(
        _prep_body,
        grid=(grid,),
        in_specs=[
            pl.BlockSpec((BN, D), lambda i: (i, 0)),
            pl.BlockSpec((D, H), lambda i: (0, 0)),
            pl.BlockSpec((1, H), lambda i: (0, 0)),
            pl.BlockSpec((H, H), lambda i: (0, 0)),
            pl.BlockSpec((1, H), lambda i: (0, 0)),
            pl.BlockSpec((H, D), lambda i: (0, 0)),
            pl.BlockSpec((1, D), lambda i: (0, 0)),
        ],
        out_specs=pl.BlockSpec((2, BN, D), lambda i: (0, i, 0)),
        out_shape=jax.ShapeDtypeStruct((2, N, D), jnp.float32),
    )(x, Wi1, bi1r, Wi2, bi2r, Wi3, bi3r)
    table2n = table.reshape(2 * N, D)

    ER = E // 128
    idx = pl.pallas_call(
        functools.partial(_idx_body, N),
        out_shape=jax.ShapeDtypeStruct((ER, 128), jnp.int32),
    )(src.reshape(ER, 128), edge_r.reshape(ER, 128)).reshape(E)

    nchunk = -(-E // (NW * CH))
    if nchunk % 2:
        nchunk += 1
    epw = nchunk * CH
    epad = NW * epw
    npad = -(-(N + 1) // 128) * 128
    pad = epad - E
    idx_p = jnp.concatenate([idx, jnp.zeros((pad,), jnp.int32)])
    dst_p = jnp.concatenate([dst, jnp.full((pad,), N, jnp.int32)])
    zrows = jnp.zeros((128, D), jnp.float32)
    orows = jnp.ones((CH, D), jnp.float32)
    zpt = npad // NS

    def zero_acc(src_stage, acc, s):
        off = 0
        while off < zpt:
            ln = min(128, zpt - off)
            pltpu.sync_copy(src_stage.at[pl.ds(0, ln)],
                            acc.at[pl.ds(s * zpt + off, ln)])
            off += ln

    def export_acc(acc, stage, out_hbm, c, s):
        off = 0
        while off < zpt:
            ln = min(128, zpt - off)
            pltpu.sync_copy(acc.at[pl.ds(s * zpt + off, ln)],
                            stage.at[pl.ds(0, ln)])
            pltpu.sync_copy(stage.at[pl.ds(0, ln)],
                            out_hbm.at[c, pl.ds(s * zpt + off, ln)])
            off += ln

    SLAB = 8
    nslab = nchunk // SLAB
    idx2d = idx_p.reshape(epad // CH, CH)
    dst2d = dst_p.reshape(epad // CH, CH)

    def sum_body(table, idx2, dst2, zrows_r, sum_out,
                 idxb, dstb, rows0, rows1, acc, sem0, sem1):
        c = lax.axis_index("c")
        s = lax.axis_index("s")
        pltpu.sync_copy(zrows_r, rows0)
        zero_acc(rows0, acc, s)
        plsc.subcore_barrier()

        wbase = (c * NS + s) * nchunk
        rows = [rows0, rows1]
        sems = [sem0, sem1]

        def slab(t, carry):
            rb = wbase + t * SLAB
            pltpu.sync_copy(idx2.at[pl.ds(rb, SLAB)], idxb)
            pltpu.sync_copy(dst2.at[pl.ds(rb, SLAB)], dstb)
            pltpu.async_copy(table.at[idxb.at[0]], rows[0], sems[0])
            for j in range(SLAB):
                if j + 1 < SLAB:
                    pltpu.async_copy(table.at[idxb.at[j + 1]],
                                     rows[(j + 1) % 2], sems[(j + 1) % 2])
                pltpu.make_async_copy(table.at[idxb.at[j]],
                                      rows[j % 2], sems[j % 2]).wait()
                pltpu.sync_copy(rows[j % 2], acc.at[dstb.at[j]], add=True)
            return carry

        lax.fori_loop(0, nslab, slab, 0, unroll=False)
        plsc.subcore_barrier()
        export_acc(acc, rows0, sum_out, c, s)

    def deg_body(dstp, zrows_r, orows_r, deg_out,
                 dst0, dst1, rows0, acc, sem0, sem1):
        c = lax.axis_index("c")
        s = lax.axis_index("s")
        pltpu.sync_copy(zrows_r, rows0)
        zero_acc(rows0, acc, s)
        plsc.subcore_barrier()
        pltpu.sync_copy(orows_r, rows0)

        ebase = (c * NS + s) * epw
        pltpu.sync_copy(dstp.at[pl.ds(ebase, CH)], dst0)

        def pair(p, carry):
            eb = ebase + 2 * p * CH
            a1 = pltpu.async_copy(dstp.at[pl.ds(eb + CH, CH)], dst1, sem1)
            pltpu.sync_copy(rows0, acc.at[dst0], add=True)
            a1.wait()
            ebn = jnp.minimum(eb + 2 * CH, ebase + (nchunk - 1) * CH)
            a0 = pltpu.async_copy(dstp.at[pl.ds(ebn, CH)], dst0, sem0)
            pltpu.sync_copy(rows0, acc.at[dst1], add=True)
            a0.wait()
            return carry

        lax.fori_loop(0, nchunk // 2, pair, 0, unroll=False)
        plsc.subcore_barrier()
        export_acc(acc, rows0, deg_out, c, s)

    mesh = plsc.VectorSubcoreMesh(
        core_axis_name="c", subcore_axis_name="s",
        num_cores=NC, num_subcores=NS)
    sum_call = pl.kernel(
        sum_body,
        out_type=jax.ShapeDtypeStruct((NC, npad, D), jnp.float32),
        mesh=mesh,
        scratch_types=[
            pltpu.VMEM((SLAB, CH), jnp.int32),
            pltpu.VMEM((SLAB, CH), jnp.int32),
            pltpu.VMEM((CH, D), jnp.float32),
            pltpu.VMEM((CH, D), jnp.float32),
            pltpu.VMEM_SHARED((npad, D), jnp.float32),
            pltpu.SemaphoreType.DMA,
            pltpu.SemaphoreType.DMA,
        ],
    )
    sum_p = sum_call(table2n, idx2d, dst2d, zrows)

    deg_call = pl.kernel(
        deg_body,
        out_type=jax.ShapeDtypeStruct((NC, npad, D), jnp.float32),
        mesh=mesh,
        scratch_types=[
            pltpu.VMEM((CH,), jnp.int32),
            pltpu.VMEM((CH,), jnp.int32),
            pltpu.VMEM((CH, D), jnp.float32),
            pltpu.VMEM_SHARED((npad, D), jnp.float32),
            pltpu.SemaphoreType.DMA,
            pltpu.SemaphoreType.DMA,
        ],
    )
    deg_p = deg_call(dst_p, zrows, orows)

    sum_p = sum_p[:, :N]
    deg_p = deg_p[:, :N, 0:1]

    inv2d = node_inv.reshape(N, 1)
    out = pl.pallas_call(
        _final_body,
        grid=(grid,),
        in_specs=[
            pl.BlockSpec((NC, BN, D), lambda i: (0, i, 0)),
            pl.BlockSpec((NC, BN, 1), lambda i: (0, i, 0)),
            pl.BlockSpec((BN, 1), lambda i: (i, 0)),
            pl.BlockSpec((D, H), lambda i: (0, 0)),
            pl.BlockSpec((1, H), lambda i: (0, 0)),
            pl.BlockSpec((H, H), lambda i: (0, 0)),
            pl.BlockSpec((1, H), lambda i: (0, 0)),
            pl.BlockSpec((H, D), lambda i: (0, 0)),
            pl.BlockSpec((1, D), lambda i: (0, 0)),
            pl.BlockSpec((D, H), lambda i: (0, 0)),
            pl.BlockSpec((1, H), lambda i: (0, 0)),
            pl.BlockSpec((H, H), lambda i: (0, 0)),
            pl.BlockSpec((1, H), lambda i: (0, 0)),
            pl.BlockSpec((H, D), lambda i: (0, 0)),
            pl.BlockSpec((1, D), lambda i: (0, 0)),
        ],
        out_specs=pl.BlockSpec((BN, D), lambda i: (i, 0)),
        out_shape=jax.ShapeDtypeStruct((N, D), jnp.float32),
    )(sum_p, deg_p, inv2d,
      Wa1, ba1r, Wa2, ba2r, Wa3, ba3r,
      Wi1, bi1r, Wi2, bi2r, Wi3, bi3r)
    return out
